# baseline (speedup 1.0000x reference)
import jax
import jax.numpy as jnp
from jax import lax
from jax.experimental import pallas as pl
from jax.experimental.pallas import tpu as pltpu
from jax.experimental.pallas import tpu_sc as plsc

_NUM_CLASSES = 1000
_N_ROWS = 16384
_NC = 2
_NS = 16
_NW = _NC * _NS
_ROWS_PER_W = _N_ROWS // _NW
_CHUNK_ROWS = 32
_N_CHUNKS = _ROWS_PER_W // _CHUNK_ROWS
_L = 16


_NBUF = 2


def _body(x_hbm, out_hbm, idx_v, buf0, buf1, sem0, sem1):
    wid = lax.axis_index("s") * _NC + lax.axis_index("c")
    base_row = wid * _ROWS_PER_W

    zvec = jnp.zeros((_L,), jnp.int32)
    onevec = jnp.full((_L,), 1, jnp.int32)
    lane = lax.iota(jnp.int32, _L)

    pltpu.sync_copy(x_hbm.at[pl.ds(base_row, _ROWS_PER_W)], idx_v)

    def _zero(r, carry):
        for buf in (buf0, buf1):
            for k in range(_NUM_CLASSES // _L):
                buf[r, pl.ds(k * _L, _L)] = zvec
            buf[r, pl.ds(_NUM_CLASSES - _L, _L)] = zvec
        return carry

    lax.fori_loop(0, _CHUNK_ROWS, _zero, 0)

    bufs = (buf0, buf1)
    sems = (sem0, sem1)
    dmas = {}
    prev_cols = {}
    for c in range(_N_CHUNKS):
        b = c % _NBUF
        buf = bufs[b]
        if c >= _NBUF:
            dmas[b].wait()
            for v, xv_old in enumerate(prev_cols[b]):
                plsc.store_scatter(buf, [lane + v * _L, xv_old], zvec)
        cols = []
        for v in range(_CHUNK_ROWS // _L):
            xv = idx_v[pl.ds(c * _CHUNK_ROWS + v * _L, _L)]
            plsc.store_scatter(buf, [lane + v * _L, xv], onevec)
            cols.append(xv)
        prev_cols[b] = cols
        row0 = base_row + c * _CHUNK_ROWS
        dma = pltpu.make_async_copy(
            buf, out_hbm.at[pl.ds(row0, _CHUNK_ROWS), :], sems[b])
        dma.start()
        dmas[b] = dma
    for b in range(_NBUF):
        dmas[b].wait()


@jax.jit
def kernel(x):
    mesh = plsc.VectorSubcoreMesh(
        core_axis_name="c", subcore_axis_name="s",
        num_cores=_NC, num_subcores=_NS)
    return pl.kernel(
        _body,
        out_type=jax.ShapeDtypeStruct((_N_ROWS, _NUM_CLASSES), jnp.int32),
        mesh=mesh,
        scratch_types=[
            pltpu.VMEM((_ROWS_PER_W,), jnp.int32),
            pltpu.VMEM((_CHUNK_ROWS, _NUM_CLASSES), jnp.int32),
            pltpu.VMEM((_CHUNK_ROWS, _NUM_CLASSES), jnp.int32),
            pltpu.SemaphoreType.DMA,
            pltpu.SemaphoreType.DMA,
        ],
        compiler_params=pltpu.CompilerParams(needs_layout_passes=False),
    )(x)
